# Initial kernel scaffold; baseline (speedup 1.0000x reference)
import jax
import jax.numpy as jnp
from jax.experimental import pallas as pl


def kernel(inputs, hidden, emb_table, Wx0, Wh0, bh0, Wy0, by0, Wx1, Wh1, bh1, Wy1, by1):
    raise NotImplementedError("write your pallas kernel here")


# speedup vs baseline: 2.8828x; 2.8828x over previous
import jax
import jax.numpy as jnp
from jax.experimental import pallas as pl
from jax.experimental.pallas import tpu as pltpu

_SEQ, _BATCH, _HID, _EMB, _VOCAB = 128, 64, 1024, 1024, 10000
_NC = 2
_BH = _BATCH // _NC
_BM = 256

_INTERPRET = False

_BF = jnp.bfloat16
_F32 = jnp.float32


def _emxu_dot(a, w_ref):
    m = a.shape[0]
    cols = []
    for b in range(4):
        mxu = b % 2
        addr = (b // 2) * (m // 4)
        for k in range(4):
            pltpu.matmul_push_rhs(w_ref[k * 256:(k + 1) * 256,
                                        b * 256:(b + 1) * 256],
                                  staging_register=0, mxu_index=mxu)
            pltpu.matmul_acc_lhs(addr, a[:, k * 256:(k + 1) * 256],
                                 mxu_index=mxu, load_staged_rhs=0)
        cols.append(pltpu.matmul_pop(addr, (m, 256), _F32, mxu_index=mxu))
    return jnp.concatenate(cols, axis=1)


def _rnn_body(emb_ref, hid_ref, wx0_ref, wh0_ref, wy0_ref, wx1_ref, wh1_ref,
              bh0_ref, by0_ref, bh1_ref, h1seq_ref, hfin_ref, h0_ref, h1_ref):
    t = pl.program_id(1)

    @pl.when(t == 0)
    def _():
        h0_ref[...] = hid_ref[0]
        h1_ref[...] = hid_ref[1]

    x = emb_ref[0]
    h0 = h0_ref[...].astype(_BF)
    h1 = h1_ref[...].astype(_BF)
    h1seq_ref[0] = h1
    y0 = (_emxu_dot(h0, wy0_ref) + by0_ref[...]).astype(_BF)
    dx = _emxu_dot(x, wx0_ref)
    dh = _emxu_dot(h0, wh0_ref)
    h0n = jnp.tanh(dx + dh + bh0_ref[...])
    dy = _emxu_dot(y0, wx1_ref)
    dh1 = _emxu_dot(h1, wh1_ref)
    h1n = jnp.tanh(dy + dh1 + bh1_ref[...])
    h0_ref[...] = h0n
    h1_ref[...] = h1n

    @pl.when(t == _SEQ - 1)
    def _():
        hfin_ref[0] = h0n
        hfin_ref[1] = h1n


def _logits_body(a_ref, w_ref, b_ref, o_ref):
    o_ref[...] = (jnp.dot(a_ref[...], w_ref[...],
                          preferred_element_type=_F32) + b_ref[...])


def _vmem():
    return pl.BlockSpec(memory_space=pltpu.VMEM)


def kernel(inputs, hidden, emb_table, Wx0, Wh0, bh0, Wy0, by0,
           Wx1, Wh1, bh1, Wy1, by1):
    emb = emb_table[inputs].astype(_BF)

    h1seq, hfin = pl.pallas_call(
        _rnn_body,
        grid=(_NC, _SEQ),
        in_specs=[
            pl.BlockSpec((1, _BH, _EMB), lambda c, t: (t, c, 0)),
            pl.BlockSpec((2, _BH, _HID), lambda c, t: (0, c, 0)),
            _vmem(), _vmem(), _vmem(), _vmem(), _vmem(),
            _vmem(), _vmem(), _vmem(),
        ],
        out_specs=[
            pl.BlockSpec((1, _BH, _HID), lambda c, t: (t, c, 0)),
            pl.BlockSpec((2, _BH, _HID), lambda c, t: (0, c, 0)),
        ],
        out_shape=[
            jax.ShapeDtypeStruct((_SEQ, _BATCH, _HID), _BF),
            jax.ShapeDtypeStruct((2, _BATCH, _HID), _F32),
        ],
        scratch_shapes=[pltpu.VMEM((_BH, _HID), _F32),
                        pltpu.VMEM((_BH, _HID), _F32)],
        compiler_params=pltpu.CompilerParams(
            dimension_semantics=("parallel", "arbitrary"),
            vmem_limit_bytes=56 * 1024 * 1024,
        ),
        name="rnn_recurrence",
        interpret=_INTERPRET,
    )(emb, hidden,
      Wx0.T.astype(_BF), Wh0.T.astype(_BF), Wy0.T.astype(_BF),
      Wx1.T.astype(_BF), Wh1.T.astype(_BF),
      bh0.reshape(1, -1), by0.reshape(1, -1), bh1.reshape(1, -1))

    a = h1seq.reshape(_SEQ * _BATCH, _HID)
    w = Wy1.T.astype(_BF)
    logits = pl.pallas_call(
        _logits_body,
        grid=(_SEQ * _BATCH // _BM,),
        in_specs=[
            pl.BlockSpec((_BM, _HID), lambda i: (i, 0)),
            _vmem(),
            _vmem(),
        ],
        out_specs=pl.BlockSpec((_BM, _VOCAB), lambda i: (i, 0)),
        out_shape=jax.ShapeDtypeStruct((_SEQ * _BATCH, _VOCAB), _F32),
        compiler_params=pltpu.CompilerParams(
            dimension_semantics=("parallel",),
            vmem_limit_bytes=56 * 1024 * 1024,
        ),
        name="rnn_logits",
        interpret=_INTERPRET,
    )(a, w, by1.reshape(1, -1))

    return logits.reshape(_SEQ, _BATCH, _VOCAB), hfin


# speedup vs baseline: 3.1078x; 1.0780x over previous
import jax
import jax.numpy as jnp
from jax.experimental import pallas as pl
from jax.experimental.pallas import tpu as pltpu

_SEQ, _BATCH, _HID, _EMB, _VOCAB = 128, 64, 1024, 1024, 10000
_NC = 2
_BH = _BATCH // _NC
_BB = 8
_BV = 2000

_INTERPRET = False

_BF = jnp.bfloat16
_F32 = jnp.float32


def _emxu_dot(a, w_ref):
    m = a.shape[0]
    cols = []
    for b in range(4):
        mxu = b % 2
        addr = (b // 2) * (m // 4)
        for k in range(4):
            pltpu.matmul_push_rhs(w_ref[k * 256:(k + 1) * 256,
                                        b * 256:(b + 1) * 256],
                                  staging_register=0, mxu_index=mxu)
            pltpu.matmul_acc_lhs(addr, a[:, k * 256:(k + 1) * 256],
                                 mxu_index=mxu, load_staged_rhs=0)
        cols.append(pltpu.matmul_pop(addr, (m, 256), _F32, mxu_index=mxu))
    return jnp.concatenate(cols, axis=1)


def _rnn_body(emb_ref, hid_ref, wx0_ref, wh0_ref, wy0_ref, wx1_ref, wh1_ref,
              bh0_ref, by0_ref, bh1_ref, h1seq_ref, hfin_ref, h0_ref, h1_ref):
    t = pl.program_id(1)

    @pl.when(t == 0)
    def _():
        h0_ref[...] = hid_ref[0]
        h1_ref[...] = hid_ref[1]

    x = emb_ref[0]
    h0 = h0_ref[...].astype(_BF)
    h1 = h1_ref[...].astype(_BF)
    h1seq_ref[0] = h1
    y0 = (_emxu_dot(h0, wy0_ref) + by0_ref[...]).astype(_BF)
    dx = _emxu_dot(x, wx0_ref)
    dh = _emxu_dot(h0, wh0_ref)
    h0n = jnp.tanh(dx + dh + bh0_ref[...])
    dy = _emxu_dot(y0, wx1_ref)
    dh1 = _emxu_dot(h1, wh1_ref)
    h1n = jnp.tanh(dy + dh1 + bh1_ref[...])
    h0_ref[...] = h0n
    h1_ref[...] = h1n

    @pl.when(t == _SEQ - 1)
    def _():
        hfin_ref[0] = h0n
        hfin_ref[1] = h1n


def _logits_body(a_ref, w_ref, b_ref, o_ref):
    w = w_ref[...]
    bias = b_ref[...]
    for bi in range(_BB):
        o_ref[bi] = jax.lax.dot_general(
            w, a_ref[:, bi, :], (((1,), (1,)), ((), ())),
            preferred_element_type=_F32) + bias


def _vmem():
    return pl.BlockSpec(memory_space=pltpu.VMEM)


def kernel(inputs, hidden, emb_table, Wx0, Wh0, bh0, Wy0, by0,
           Wx1, Wh1, bh1, Wy1, by1):
    emb = emb_table[inputs].astype(_BF)

    h1seq, hfin = pl.pallas_call(
        _rnn_body,
        grid=(_NC, _SEQ),
        in_specs=[
            pl.BlockSpec((1, _BH, _EMB), lambda c, t: (t, c, 0)),
            pl.BlockSpec((2, _BH, _HID), lambda c, t: (0, c, 0)),
            _vmem(), _vmem(), _vmem(), _vmem(), _vmem(),
            _vmem(), _vmem(), _vmem(),
        ],
        out_specs=[
            pl.BlockSpec((1, _BH, _HID), lambda c, t: (t, c, 0)),
            pl.BlockSpec((2, _BH, _HID), lambda c, t: (0, c, 0)),
        ],
        out_shape=[
            jax.ShapeDtypeStruct((_SEQ, _BATCH, _HID), _BF),
            jax.ShapeDtypeStruct((2, _BATCH, _HID), _F32),
        ],
        scratch_shapes=[pltpu.VMEM((_BH, _HID), _F32),
                        pltpu.VMEM((_BH, _HID), _F32)],
        compiler_params=pltpu.CompilerParams(
            dimension_semantics=("parallel", "arbitrary"),
            vmem_limit_bytes=56 * 1024 * 1024,
        ),
        name="rnn_recurrence",
        interpret=_INTERPRET,
    )(emb, hidden,
      Wx0.T.astype(_BF), Wh0.T.astype(_BF), Wy0.T.astype(_BF),
      Wx1.T.astype(_BF), Wh1.T.astype(_BF),
      bh0.reshape(1, -1), by0.reshape(1, -1), bh1.reshape(1, -1))

    w = Wy1.astype(_BF)
    logits_bvt = pl.pallas_call(
        _logits_body,
        grid=(_BATCH // _BB, _VOCAB // _BV),
        in_specs=[
            pl.BlockSpec((_SEQ, _BB, _HID), lambda b, v: (0, b, 0)),
            pl.BlockSpec((_BV, _HID), lambda b, v: (v, 0)),
            pl.BlockSpec((_BV, 1), lambda b, v: (v, 0)),
        ],
        out_specs=pl.BlockSpec((_BB, _BV, _SEQ), lambda b, v: (b, v, 0)),
        out_shape=jax.ShapeDtypeStruct((_BATCH, _VOCAB, _SEQ), _F32),
        compiler_params=pltpu.CompilerParams(
            dimension_semantics=("parallel", "arbitrary"),
            vmem_limit_bytes=56 * 1024 * 1024,
        ),
        name="rnn_logits",
        interpret=_INTERPRET,
    )(h1seq, w, by1.reshape(-1, 1))

    return jnp.transpose(logits_bvt, (2, 0, 1)), hfin


# speedup vs baseline: 4.1640x; 1.3399x over previous
import jax
import jax.numpy as jnp
from jax.experimental import pallas as pl
from jax.experimental.pallas import tpu as pltpu

_SEQ, _BATCH, _HID, _EMB, _VOCAB = 128, 64, 1024, 1024, 10000
_NC = 2
_BH = _BATCH // _NC
_BB = 8
_BV = 2000

_INTERPRET = False

_BF = jnp.bfloat16
_F32 = jnp.float32


def _emxu_dot(a, w_ref):
    m = a.shape[0]
    cols = []
    for b in range(4):
        mxu = b % 2
        addr = (b // 2) * (m // 4)
        for k in range(4):
            pltpu.matmul_push_rhs(w_ref[k * 256:(k + 1) * 256,
                                        b * 256:(b + 1) * 256],
                                  staging_register=0, mxu_index=mxu)
            pltpu.matmul_acc_lhs(addr, a[:, k * 256:(k + 1) * 256],
                                 mxu_index=mxu, load_staged_rhs=0)
        cols.append(pltpu.matmul_pop(addr, (m, 256), _F32, mxu_index=mxu))
    return jnp.concatenate(cols, axis=1)


def _rnn_body(emb_ref, hid_ref, wx0_ref, wh0_ref, wy0_ref, wx1_ref, wh1_ref,
              bh0_ref, by0_ref, bh1_ref, h1seq_ref, hfin_ref, h0_ref, h1_ref):
    t = pl.program_id(0)

    @pl.when(t == 0)
    def _():
        h0_ref[...] = hid_ref[0]
        h1_ref[...] = hid_ref[1]

    x = emb_ref[0]
    h0 = h0_ref[...].astype(_BF)
    h1 = h1_ref[...].astype(_BF)
    h1seq_ref[0] = h1
    y0 = (_emxu_dot(h0, wy0_ref) + by0_ref[...]).astype(_BF)
    dx = _emxu_dot(x, wx0_ref)
    dh = _emxu_dot(h0, wh0_ref)
    h0n = jnp.tanh(dx + dh + bh0_ref[...])
    dy = _emxu_dot(y0, wx1_ref)
    dh1 = _emxu_dot(h1, wh1_ref)
    h1n = jnp.tanh(dy + dh1 + bh1_ref[...])
    h0_ref[...] = h0n
    h1_ref[...] = h1n

    @pl.when(t == _SEQ - 1)
    def _():
        hfin_ref[0] = h0n
        hfin_ref[1] = h1n


def _logits_body(a_ref, w_ref, b_ref, o_ref):
    w = w_ref[...]
    bias = b_ref[...]
    for bi in range(_BB):
        o_ref[bi] = jax.lax.dot_general(
            w, a_ref[:, bi, :], (((1,), (1,)), ((), ())),
            preferred_element_type=_F32) + bias


def _vmem():
    return pl.BlockSpec(memory_space=pltpu.VMEM)


def kernel(inputs, hidden, emb_table, Wx0, Wh0, bh0, Wy0, by0,
           Wx1, Wh1, bh1, Wy1, by1):
    emb = emb_table[inputs].astype(_BF)

    h1seq, hfin = pl.pallas_call(
        _rnn_body,
        grid=(_SEQ,),
        in_specs=[
            pl.BlockSpec((1, _BATCH, _EMB), lambda t: (t, 0, 0)),
            pl.BlockSpec((2, _BATCH, _HID), lambda t: (0, 0, 0)),
            _vmem(), _vmem(), _vmem(), _vmem(), _vmem(),
            _vmem(), _vmem(), _vmem(),
        ],
        out_specs=[
            pl.BlockSpec((1, _BATCH, _HID), lambda t: (t, 0, 0)),
            pl.BlockSpec((2, _BATCH, _HID), lambda t: (0, 0, 0)),
        ],
        out_shape=[
            jax.ShapeDtypeStruct((_SEQ, _BATCH, _HID), _BF),
            jax.ShapeDtypeStruct((2, _BATCH, _HID), _F32),
        ],
        scratch_shapes=[pltpu.VMEM((_BATCH, _HID), _F32),
                        pltpu.VMEM((_BATCH, _HID), _F32)],
        compiler_params=pltpu.CompilerParams(
            dimension_semantics=("arbitrary",),
            vmem_limit_bytes=56 * 1024 * 1024,
        ),
        name="rnn_recurrence",
        interpret=_INTERPRET,
    )(emb, hidden,
      Wx0.T.astype(_BF), Wh0.T.astype(_BF), Wy0.T.astype(_BF),
      Wx1.T.astype(_BF), Wh1.T.astype(_BF),
      bh0.reshape(1, -1), by0.reshape(1, -1), bh1.reshape(1, -1))

    w = Wy1.astype(_BF)
    logits_bvt = pl.pallas_call(
        _logits_body,
        grid=(_BATCH // _BB, _VOCAB // _BV),
        in_specs=[
            pl.BlockSpec((_SEQ, _BB, _HID), lambda b, v: (0, b, 0)),
            pl.BlockSpec((_BV, _HID), lambda b, v: (v, 0)),
            pl.BlockSpec((_BV, 1), lambda b, v: (v, 0)),
        ],
        out_specs=pl.BlockSpec((_BB, _BV, _SEQ), lambda b, v: (b, v, 0)),
        out_shape=jax.ShapeDtypeStruct((_BATCH, _VOCAB, _SEQ), _F32),
        compiler_params=pltpu.CompilerParams(
            dimension_semantics=("arbitrary", "arbitrary"),
            vmem_limit_bytes=56 * 1024 * 1024,
        ),
        name="rnn_logits",
        interpret=_INTERPRET,
    )(h1seq, w, by1.reshape(-1, 1))

    return jnp.transpose(logits_bvt, (2, 0, 1)), hfin


# speedup vs baseline: 4.6833x; 1.1247x over previous
import jax
import jax.numpy as jnp
from jax.experimental import pallas as pl
from jax.experimental.pallas import tpu as pltpu

_SEQ, _BATCH, _HID, _EMB, _VOCAB = 128, 64, 1024, 1024, 10000
_NC = 2
_BH = _BATCH // _NC
_BB = 8
_BV = 2000

_INTERPRET = False

_BF = jnp.bfloat16
_F32 = jnp.float32


def _emxu_dot(a, w_ref):
    m = a.shape[0]
    cols = []
    for b in range(4):
        mxu = b % 2
        addr = (b // 2) * (m // 4)
        for k in range(4):
            pltpu.matmul_push_rhs(w_ref[k * 256:(k + 1) * 256,
                                        b * 256:(b + 1) * 256],
                                  staging_register=0, mxu_index=mxu)
            pltpu.matmul_acc_lhs(addr, a[:, k * 256:(k + 1) * 256],
                                 mxu_index=mxu, load_staged_rhs=0)
        cols.append(pltpu.matmul_pop(addr, (m, 256), _F32, mxu_index=mxu))
    return jnp.concatenate(cols, axis=1)


def _rnn_body(emb_ref, hid_ref, wx0_ref, wh0_ref, wy0_ref, wx1_ref, wh1_ref,
              bh0_ref, by0_ref, bh1_ref, h1seq_ref, hfin_ref, h0_ref, h1_ref):
    t = pl.program_id(0)

    @pl.when(t == 0)
    def _():
        h0_ref[...] = hid_ref[0]
        h1_ref[...] = hid_ref[1]

    x = emb_ref[0]
    h0 = h0_ref[...].astype(_BF)
    h1 = h1_ref[...].astype(_BF)
    h1seq_ref[0] = h1
    y0 = (_emxu_dot(h0, wy0_ref) + by0_ref[...]).astype(_BF)
    dx = _emxu_dot(x, wx0_ref)
    dh = _emxu_dot(h0, wh0_ref)
    h0n = jnp.tanh(dx + dh + bh0_ref[...])
    dy = _emxu_dot(y0, wx1_ref)
    dh1 = _emxu_dot(h1, wh1_ref)
    h1n = jnp.tanh(dy + dh1 + bh1_ref[...])
    h0_ref[...] = h0n
    h1_ref[...] = h1n

    @pl.when(t == _SEQ - 1)
    def _():
        hfin_ref[0] = h0n
        hfin_ref[1] = h1n


def _logits_body(a_ref, w_ref, b_ref, o_ref):
    bias = b_ref[...]
    for p in range(_BB // 2):
        mxu = p % 2
        tiles = []
        for k in range(4):
            ks = slice(k * 256, (k + 1) * 256)
            tiles.append(jnp.concatenate(
                [a_ref[:, 2 * p, ks], a_ref[:, 2 * p + 1, ks]], axis=0))
        for m0, m1 in ((0, 1008), (1008, _BV)):
            for k in range(4):
                ks = slice(k * 256, (k + 1) * 256)
                pltpu.matmul_push_rhs(tiles[k], staging_register=0,
                                      mxu_index=mxu, transpose=True)
                pltpu.matmul_acc_lhs(0, w_ref[m0:m1, ks], mxu_index=mxu,
                                     load_staged_rhs=0)
            res = pltpu.matmul_pop(0, (m1 - m0, 256), _F32, mxu_index=mxu)
            o_ref[2 * p, m0:m1, :] = res[:, :_SEQ] + bias[m0:m1]
            o_ref[2 * p + 1, m0:m1, :] = res[:, _SEQ:] + bias[m0:m1]


def _vmem():
    return pl.BlockSpec(memory_space=pltpu.VMEM)


def kernel(inputs, hidden, emb_table, Wx0, Wh0, bh0, Wy0, by0,
           Wx1, Wh1, bh1, Wy1, by1):
    emb = emb_table[inputs].astype(_BF)

    h1seq, hfin = pl.pallas_call(
        _rnn_body,
        grid=(_SEQ,),
        in_specs=[
            pl.BlockSpec((1, _BATCH, _EMB), lambda t: (t, 0, 0)),
            pl.BlockSpec((2, _BATCH, _HID), lambda t: (0, 0, 0)),
            _vmem(), _vmem(), _vmem(), _vmem(), _vmem(),
            _vmem(), _vmem(), _vmem(),
        ],
        out_specs=[
            pl.BlockSpec((1, _BATCH, _HID), lambda t: (t, 0, 0)),
            pl.BlockSpec((2, _BATCH, _HID), lambda t: (0, 0, 0)),
        ],
        out_shape=[
            jax.ShapeDtypeStruct((_SEQ, _BATCH, _HID), _BF),
            jax.ShapeDtypeStruct((2, _BATCH, _HID), _F32),
        ],
        scratch_shapes=[pltpu.VMEM((_BATCH, _HID), _F32),
                        pltpu.VMEM((_BATCH, _HID), _F32)],
        compiler_params=pltpu.CompilerParams(
            dimension_semantics=("arbitrary",),
            vmem_limit_bytes=56 * 1024 * 1024,
        ),
        name="rnn_recurrence",
        interpret=_INTERPRET,
    )(emb, hidden,
      Wx0.T.astype(_BF), Wh0.T.astype(_BF), Wy0.T.astype(_BF),
      Wx1.T.astype(_BF), Wh1.T.astype(_BF),
      bh0.reshape(1, -1), by0.reshape(1, -1), bh1.reshape(1, -1))

    w = Wy1.astype(_BF)
    logits_bvt = pl.pallas_call(
        _logits_body,
        grid=(_BATCH // _BB, _VOCAB // _BV),
        in_specs=[
            pl.BlockSpec((_SEQ, _BB, _HID), lambda b, v: (0, b, 0)),
            pl.BlockSpec((_BV, _HID), lambda b, v: (v, 0)),
            pl.BlockSpec((_BV, 1), lambda b, v: (v, 0)),
        ],
        out_specs=pl.BlockSpec((_BB, _BV, _SEQ), lambda b, v: (b, v, 0)),
        out_shape=jax.ShapeDtypeStruct((_BATCH, _VOCAB, _SEQ), _F32),
        compiler_params=pltpu.CompilerParams(
            dimension_semantics=("arbitrary", "arbitrary"),
            vmem_limit_bytes=56 * 1024 * 1024,
        ),
        name="rnn_logits",
        interpret=_INTERPRET,
    )(h1seq, w, by1.reshape(-1, 1))

    return jnp.transpose(logits_bvt, (2, 0, 1)), hfin


# speedup vs baseline: 4.7019x; 1.0040x over previous
import jax
import jax.numpy as jnp
from jax.experimental import pallas as pl
from jax.experimental.pallas import tpu as pltpu

_SEQ, _BATCH, _HID, _EMB, _VOCAB = 128, 64, 1024, 1024, 10000
_NC = 2
_BH = _BATCH // _NC
_BB = 8
_BV = 2000

_INTERPRET = False

_BF = jnp.bfloat16
_F32 = jnp.float32


def _emxu_dot(a, w_ref):
    m = a.shape[0]
    cols = []
    for b in range(4):
        mxu = b % 2
        addr = (b // 2) * (m // 4)
        for k in range(4):
            pltpu.matmul_push_rhs(w_ref[k * 256:(k + 1) * 256,
                                        b * 256:(b + 1) * 256],
                                  staging_register=0, mxu_index=mxu)
            pltpu.matmul_acc_lhs(addr, a[:, k * 256:(k + 1) * 256],
                                 mxu_index=mxu, load_staged_rhs=0)
        cols.append(pltpu.matmul_pop(addr, (m, 256), _F32, mxu_index=mxu))
    return jnp.concatenate(cols, axis=1)


def _rnn_body(emb_ref, hid_ref, wx0_ref, wh0_ref, wy0_ref, wx1_ref, wh1_ref,
              bh0_ref, by0_ref, bh1_ref, h1seq_ref, hfin_ref, h0_ref, h1_ref):
    t = pl.program_id(0)

    @pl.when(t == 0)
    def _():
        h0_ref[...] = hid_ref[0]
        h1_ref[...] = hid_ref[1]

    x = emb_ref[0]
    h0 = h0_ref[...].astype(_BF)
    h1 = h1_ref[...].astype(_BF)
    h1seq_ref[0] = h1
    y0 = (_emxu_dot(h0, wy0_ref) + by0_ref[...]).astype(_BF)
    dx = _emxu_dot(x, wx0_ref)
    dh = _emxu_dot(h0, wh0_ref)
    h0n = jnp.tanh(dx + dh + bh0_ref[...])
    dy = _emxu_dot(y0, wx1_ref)
    dh1 = _emxu_dot(h1, wh1_ref)
    h1n = jnp.tanh(dy + dh1 + bh1_ref[...])
    h0_ref[...] = h0n
    h1_ref[...] = h1n

    @pl.when(t == _SEQ - 1)
    def _():
        hfin_ref[0] = h0n
        hfin_ref[1] = h1n


def _logits_body(a_ref, w_ref, b_ref, o_ref):
    bias = b_ref[...]
    for p in range(_BB // 2):
        mxu = p % 2
        tiles = []
        for k in range(4):
            ks = slice(k * 256, (k + 1) * 256)
            tiles.append(jnp.concatenate(
                [a_ref[:, 2 * p, ks], a_ref[:, 2 * p + 1, ks]], axis=0))
        for m0, m1 in ((0, 1008), (1008, _BV)):
            for k in range(4):
                ks = slice(k * 256, (k + 1) * 256)
                pltpu.matmul_push_rhs(tiles[k], staging_register=0,
                                      mxu_index=mxu, transpose=True)
                pltpu.matmul_acc_lhs(0, w_ref[m0:m1, ks], mxu_index=mxu,
                                     load_staged_rhs=0)
            res = pltpu.matmul_pop(0, (m1 - m0, 256), _F32, mxu_index=mxu)
            o_ref[2 * p, m0:m1, :] = res[:, :_SEQ] + bias[m0:m1]
            o_ref[2 * p + 1, m0:m1, :] = res[:, _SEQ:] + bias[m0:m1]


def _vmem():
    return pl.BlockSpec(memory_space=pltpu.VMEM)


def kernel(inputs, hidden, emb_table, Wx0, Wh0, bh0, Wy0, by0,
           Wx1, Wh1, bh1, Wy1, by1):
    emb = emb_table[inputs].astype(_BF)

    h1seq, hfin = pl.pallas_call(
        _rnn_body,
        grid=(_SEQ,),
        in_specs=[
            pl.BlockSpec((1, _BATCH, _EMB), lambda t: (t, 0, 0)),
            pl.BlockSpec((2, _BATCH, _HID), lambda t: (0, 0, 0)),
            _vmem(), _vmem(), _vmem(), _vmem(), _vmem(),
            _vmem(), _vmem(), _vmem(),
        ],
        out_specs=[
            pl.BlockSpec((1, _BATCH, _HID), lambda t: (t, 0, 0)),
            pl.BlockSpec((2, _BATCH, _HID), lambda t: (0, 0, 0)),
        ],
        out_shape=[
            jax.ShapeDtypeStruct((_SEQ, _BATCH, _HID), _BF),
            jax.ShapeDtypeStruct((2, _BATCH, _HID), _F32),
        ],
        scratch_shapes=[pltpu.VMEM((_BATCH, _HID), _F32),
                        pltpu.VMEM((_BATCH, _HID), _F32)],
        compiler_params=pltpu.CompilerParams(
            dimension_semantics=("arbitrary",),
            vmem_limit_bytes=56 * 1024 * 1024,
        ),
        name="rnn_recurrence",
        interpret=_INTERPRET,
    )(emb, hidden,
      Wx0.T.astype(_BF), Wh0.T.astype(_BF), Wy0.T.astype(_BF),
      Wx1.T.astype(_BF), Wh1.T.astype(_BF),
      bh0.reshape(1, -1), by0.reshape(1, -1), bh1.reshape(1, -1))

    w = Wy1.astype(_BF)
    logits_bvt = pl.pallas_call(
        _logits_body,
        grid=(_VOCAB // _BV, _BATCH // _BB),
        in_specs=[
            pl.BlockSpec((_SEQ, _BB, _HID), lambda v, b: (0, b, 0)),
            pl.BlockSpec((_BV, _HID), lambda v, b: (v, 0)),
            pl.BlockSpec((_BV, 1), lambda v, b: (v, 0)),
        ],
        out_specs=pl.BlockSpec((_BB, _BV, _SEQ), lambda v, b: (b, v, 0)),
        out_shape=jax.ShapeDtypeStruct((_BATCH, _VOCAB, _SEQ), _F32),
        compiler_params=pltpu.CompilerParams(
            dimension_semantics=("arbitrary", "arbitrary"),
            vmem_limit_bytes=56 * 1024 * 1024,
        ),
        name="rnn_logits",
        interpret=_INTERPRET,
    )(h1seq, w, by1.reshape(-1, 1))

    return jnp.transpose(logits_bvt, (2, 0, 1)), hfin
